# Initial kernel scaffold; baseline (speedup 1.0000x reference)
import jax
import jax.numpy as jnp
from jax.experimental import pallas as pl


def kernel(x, edge_index, W1, b1, W2, b2):
    raise NotImplementedError("write your pallas kernel here")


# speedup vs baseline: 18.7359x; 18.7359x over previous
import functools

import jax
import jax.numpy as jnp
from jax import lax
from jax.experimental import pallas as pl
from jax.experimental.pallas import tpu as pltpu
from jax.experimental.pallas import tpu_sc as plsc

NC = 2
NS = 16
NW = NC * NS
B = 128

f32 = jnp.float32


def _mesh():
    return plsc.VectorSubcoreMesh(
        core_axis_name="c", subcore_axis_name="s", num_cores=NC, num_subcores=NS
    )


def _deg_call(dst_r, zeros, ones, n_pad, ch):
    rps = n_pad // NS

    @functools.partial(
        pl.kernel,
        out_type=jax.ShapeDtypeStruct((NC, n_pad, 16), f32),
        mesh=_mesh(),
        scratch_types=[
            pltpu.VMEM((ch, B), jnp.int32),
            pltpu.VMEM((B, 16), f32),
            pltpu.VMEM_SHARED((n_pad, 16), f32),
        ],
        compiler_params=pltpu.CompilerParams(use_tc_tiling_on_sc=False),
    )
    def k(dst_hbm, zeros_hbm, ones_hbm, out_hbm, dst_v, ones_v, acc_sh):
        c = lax.axis_index("c")
        s = lax.axis_index("s")
        w = c * NS + s
        pltpu.sync_copy(zeros_hbm.at[pl.ds(s * rps, rps)],
                        acc_sh.at[pl.ds(s * rps, rps)])
        pltpu.sync_copy(ones_hbm, ones_v)
        pltpu.sync_copy(dst_hbm.at[w], dst_v)
        plsc.subcore_barrier()

        def body(j, carry):
            pltpu.sync_copy(ones_v, acc_sh.at[dst_v.at[j]], add=True)
            return carry

        lax.fori_loop(0, ch, body, 0)
        plsc.subcore_barrier()
        pltpu.sync_copy(acc_sh.at[pl.ds(s * rps, rps)],
                        out_hbm.at[c, pl.ds(s * rps, rps)])

    return k(dst_r, zeros, ones)


def _agg_call(src_r, dst_r, table, n_pad, ch, w_feat):
    rps = n_pad // NS

    @functools.partial(
        pl.kernel,
        out_type=jax.ShapeDtypeStruct((NC, n_pad, w_feat), f32),
        mesh=_mesh(),
        scratch_types=[
            pltpu.VMEM((ch, B), jnp.int32),
            pltpu.VMEM((ch, B), jnp.int32),
            pltpu.VMEM((B, w_feat), f32),
            pltpu.VMEM_SHARED((n_pad, w_feat), f32),
            pltpu.SemaphoreType.DMA,
        ],
        compiler_params=pltpu.CompilerParams(use_tc_tiling_on_sc=False),
    )
    def k(src_hbm, dst_hbm, table_hbm, out_hbm, src_v, dst_v, rows_v, acc_sh, gsem):
        c = lax.axis_index("c")
        s = lax.axis_index("s")
        w = c * NS + s
        pltpu.sync_copy(table_hbm.at[pl.ds(s * rps, rps)],
                        acc_sh.at[pl.ds(s * rps, rps)])
        pltpu.sync_copy(src_hbm.at[w], src_v)
        pltpu.sync_copy(dst_hbm.at[w], dst_v)
        plsc.subcore_barrier()

        def body(j, carry):
            pltpu.async_copy(table_hbm.at[src_v.at[j]], rows_v, gsem).wait()
            pltpu.sync_copy(rows_v, acc_sh.at[dst_v.at[j]], add=True)
            return carry

        lax.fori_loop(0, ch, body, 0)
        plsc.subcore_barrier()
        pltpu.sync_copy(acc_sh.at[pl.ds(s * rps, rps)],
                        out_hbm.at[c, pl.ds(s * rps, rps)])

    return k(src_r, dst_r, table)


def _tc_b_call(x_pad, w1, b1, degs, n_pad, hid):
    def body(x_ref, w_ref, b_ref, deg_ref, t1_ref, norm_ref):
        deg = deg_ref[0] + deg_ref[1]
        norm = lax.rsqrt(deg[:, 0:1] + 1.0)
        h1 = jnp.dot(x_ref[...], w_ref[...],
                     preferred_element_type=f32) + b_ref[...]
        t1_ref[...] = h1 * norm
        norm_ref[...] = jnp.broadcast_to(norm, (n_pad, 16))

    return pl.pallas_call(
        body,
        out_shape=(
            jax.ShapeDtypeStruct((n_pad, hid), f32),
            jax.ShapeDtypeStruct((n_pad, 16), f32),
        ),
    )(x_pad, w1, b1, degs)


def _tc_c_call(acc1, t1, norm16, w2, b2, n_pad, f_out):
    def body(acc_ref, t1_ref, norm_ref, w_ref, b_ref, t2_ref):
        norm = norm_ref[:, 0:1]
        agg1 = norm * (acc_ref[0] + acc_ref[1] - t1_ref[...])
        g = jnp.where(agg1 > 0.0,
                      agg1, jnp.exp(jnp.minimum(agg1, 0.0)) - 1.0)
        h2 = jnp.dot(g, w_ref[...], preferred_element_type=f32) + b_ref[...]
        t2_ref[...] = h2 * norm

    return pl.pallas_call(
        body,
        out_shape=jax.ShapeDtypeStruct((n_pad, f_out), f32),
    )(acc1, t1, norm16, w2, b2)


def _tc_d_call(acc2, t2, norm16, n, f_out):
    def body(acc_ref, t2_ref, norm_ref, out_ref):
        norm = norm_ref[:, 0:1]
        agg2 = norm * (acc_ref[0] + acc_ref[1] - t2_ref[...])
        a = agg2[:n]
        m = jnp.max(a, axis=1, keepdims=True)
        lse = jnp.log(jnp.sum(jnp.exp(a - m), axis=1, keepdims=True))
        out_ref[...] = a - m - lse

    return pl.pallas_call(
        body,
        out_shape=jax.ShapeDtypeStruct((n, f_out), f32),
    )(acc2, t2, norm16)


def kernel(x, edge_index, W1, b1, W2, b2):
    n, f_in = x.shape
    hid = W1.shape[1]
    f_out = W2.shape[1]
    e = edge_index.shape[1]

    align = NS * 8
    n_pad = ((n + 1 + align - 1) // align) * align
    ch = -(-e // (NW * B))
    e_pad = NW * ch * B

    pad = jnp.full((e_pad - e,), n, dtype=jnp.int32)
    src_r = jnp.concatenate([edge_index[0], pad]).reshape(NW, ch, B)
    dst_r = jnp.concatenate([edge_index[1], pad]).reshape(NW, ch, B)

    x_pad = jnp.zeros((n_pad, f_in), f32).at[:n].set(x)
    zeros = jnp.zeros((n_pad, 16), f32)
    ones = jnp.ones((B, 16), f32)

    degs = _deg_call(dst_r, zeros, ones, n_pad, ch)
    t1, norm16 = _tc_b_call(x_pad, W1, b1.reshape(1, hid), degs, n_pad, hid)
    acc1 = _agg_call(src_r, dst_r, t1, n_pad, ch, hid)
    t2 = _tc_c_call(acc1, t1, norm16, W2, b2.reshape(1, f_out), n_pad, f_out)
    acc2 = _agg_call(src_r, dst_r, t2, n_pad, ch, f_out)
    return _tc_d_call(acc2, t2, norm16, n, f_out)
